# baseline (speedup 1.0000x reference)
import functools

import jax
import jax.numpy as jnp
from jax.experimental import pallas as pl
from jax.experimental.pallas import tpu as pltpu

_K = 5
_CIN = 3
_SP = 1024
_LANE_PAD = 1184
_CONV1_N = 32
_G = 8
_P1_N = 224
_S2 = 16
_CONV2_L = 160
_COUT2 = 128
_FC_SP = 25
_FC_H = 128


def _fused_body(x_ref, w1_ref, p1_ref, w2_ref, b2t_ref, pm_ref,
                wf1_ref, bf1_ref, wf2_ref, bf2_ref, wf3_ref, bf3_ref,
                o_ref, *, bb):
    f32, bf16 = jnp.float32, jnp.bfloat16
    xall = x_ref[...].astype(bf16)
    w1b, p1 = w1_ref[...], p1_ref[...]
    ones_row = jnp.ones((1, 1152), bf16)

    a1_parts = []
    for bi in range(bb):
        xip = jnp.concatenate(
            [xall[bi], jnp.zeros((_CIN, _LANE_PAD - _SP), bf16)], axis=1)
        xd = jnp.concatenate(
            [xip[:, dj:dj + 1152] for dj in range(_K)] + [ones_row],
            axis=0)
        xt = jnp.concatenate(
            [xd[:, di * 32:di * 32 + _SP] for di in range(_K)],
            axis=0)
        acc = jnp.dot(w1b, xt, preferred_element_type=f32)
        a1_parts.append(jnp.tanh(acc).astype(bf16))

    a1s = jnp.concatenate(a1_parts, axis=0)
    pooled1 = jnp.dot(a1s, p1, preferred_element_type=f32).astype(bf16)
    pooled1 = jnp.concatenate(
        [pooled1, jnp.zeros((bb * _CONV1_N, 256 - _P1_N), bf16)],
        axis=1)

    x2 = jnp.concatenate(
        [jnp.concatenate(
            [pooled1[bi * _CONV1_N:(bi + 1) * _CONV1_N,
                     di * _S2 + dj:di * _S2 + dj + _CONV2_L]
             for bi in range(bb)], axis=1)
         for di in range(_K) for dj in range(_K)], axis=0)
    acc2 = jnp.dot(w2_ref[...], x2, preferred_element_type=f32)
    a2_all = jnp.tanh(acc2 + b2t_ref[...]).astype(bf16)

    pooled2 = jnp.dot(a2_all, pm_ref[...], preferred_element_type=f32)
    pooled2 = jnp.transpose(pooled2)

    p2f = jnp.concatenate(
        [pooled2[p * bb:(p + 1) * bb, :] for p in range(_FC_SP)],
        axis=1).astype(bf16)
    h = jnp.dot(p2f, wf1_ref[...], preferred_element_type=f32)
    h = jnp.tanh(h + bf1_ref[...]).astype(bf16)

    h = jnp.tanh(jnp.dot(h, wf2_ref[...], preferred_element_type=f32)
                 + bf2_ref[...]).astype(bf16)
    o = jnp.dot(h, wf3_ref[...], preferred_element_type=f32) + bf3_ref[...]
    o_ref[...] = o.reshape(bb, 1, 2)


def kernel(x, w1, b1, p1, w2, b2, p2, wf1, bf1, wf2, bf2, wf3, bf3):
    B = x.shape[0]
    bb = 1
    for cand in (32, 16, 8, 4, 2):
        if B % cand == 0 and B // cand >= 2:
            bb = cand
            break
    bf16 = jnp.bfloat16

    xr = x.reshape(B, _CIN, _SP)

    w1r = w1.reshape(_K, _K, 8, _CONV1_N)[:, :, :_CIN, :]
    w1t = jnp.transpose(w1r, (3, 0, 1, 2)).reshape(_CONV1_N, _K, _K * _CIN)
    bcol = jnp.concatenate(
        [jnp.transpose(b1)] + [jnp.zeros((_CONV1_N, 1), b1.dtype)] * (_K - 1),
        axis=1).astype(w1t.dtype)
    w1b = jnp.concatenate([w1t, bcol[:, :, None]], axis=2)
    w1b = w1b.reshape(_CONV1_N, _K * 16)

    p1t = jnp.pad(jnp.transpose(p1[:_P1_N]), ((0, _SP - 896), (0, 0)))
    p1t = p1t.astype(bf16)

    w2t = jnp.transpose(w2)
    b2t = jnp.transpose(b2)

    eye = jnp.eye(bb, dtype=p2.dtype)
    pm = (eye[:, None, None, :] * jnp.transpose(p2)[None, :, :, None])
    pm = pm.reshape(bb * _CONV2_L, 32 * bb)

    wf1s = wf1.reshape(_FC_SP * _FC_H, _FC_H)

    body = functools.partial(_fused_body, bb=bb)
    c2 = lambda i: (0, 0)

    out = pl.pallas_call(
        body,
        grid=(B // bb,),
        out_shape=jax.ShapeDtypeStruct((B, 1, 2), jnp.float32),
        in_specs=[
            pl.BlockSpec((bb, _CIN, _SP), lambda i: (i, 0, 0)),
            pl.BlockSpec((_CONV1_N, _K * 16), c2),
            pl.BlockSpec((_SP, _P1_N), c2),
            pl.BlockSpec((_COUT2, _K * _K * _CONV1_N), c2),
            pl.BlockSpec((_COUT2, 1), c2),
            pl.BlockSpec((bb * _CONV2_L, 32 * bb), c2),
            pl.BlockSpec((_FC_SP * _FC_H, _FC_H), c2),
            pl.BlockSpec((1, _FC_H), c2),
            pl.BlockSpec((_FC_H, _FC_H), c2),
            pl.BlockSpec((1, _FC_H), c2),
            pl.BlockSpec((_FC_H, 2), c2),
            pl.BlockSpec((1, 2), c2),
        ],
        out_specs=pl.BlockSpec((bb, 1, 2), lambda i: (i, 0, 0)),
        compiler_params=pltpu.CompilerParams(
            dimension_semantics=("parallel",)),
    )(xr, w1b, p1t, w2t, b2t, pm,
      wf1s, bf1, wf2, bf2, wf3, bf3)
    return out.reshape(B, 2)


# speedup vs baseline: 1.0234x; 1.0234x over previous
import functools

import jax
import jax.numpy as jnp
from jax.experimental import pallas as pl
from jax.experimental.pallas import tpu as pltpu

_K = 5
_CIN = 3
_SP = 1024
_LANE_PAD = 1184
_CONV1_N = 32
_G = 8
_P1_N = 224
_S2 = 16
_CONV2_L = 160
_COUT2 = 128
_FC_SP = 25
_FC_H = 128


def _fused_body(x_ref, w1_ref, b1_ref, p1_ref, w2_ref, b2t_ref, pm_ref,
                wf1_ref, bf1_ref, wf2_ref, bf2_ref, wf3_ref, bf3_ref,
                o_ref, *, bb):
    f32, bf16 = jnp.float32, jnp.bfloat16
    xall = x_ref[...].astype(bf16)
    w1b, b1s, p1 = w1_ref[...], b1_ref[...], p1_ref[...]

    a1_parts = []
    for bi in range(bb):
        xip = jnp.concatenate(
            [xall[bi], jnp.zeros((_CIN, _LANE_PAD - _SP), bf16)], axis=1)
        xd = jnp.concatenate(
            [xip[:, dj:dj + 1152] for dj in range(_K)], axis=0)
        xdp = jnp.concatenate([xd, jnp.zeros((1, 1152), bf16)], axis=0)
        xt = jnp.concatenate(
            [xdp[:, di * 32:di * 32 + _SP] for di in range(_K)],
            axis=0)
        acc = jnp.dot(w1b, xt, preferred_element_type=f32)
        a1_parts.append(jnp.tanh(acc + b1s).astype(bf16))

    a1s = jnp.concatenate(a1_parts, axis=0)
    pooled1 = jnp.dot(a1s, p1, preferred_element_type=f32).astype(bf16)
    pooled1 = jnp.concatenate(
        [pooled1, jnp.zeros((bb * _CONV1_N, 256 - _P1_N), bf16)],
        axis=1)

    x2 = jnp.concatenate(
        [jnp.concatenate(
            [pooled1[bi * _CONV1_N:(bi + 1) * _CONV1_N,
                     di * _S2 + dj:di * _S2 + dj + _CONV2_L]
             for bi in range(bb)], axis=1)
         for di in range(_K) for dj in range(_K)], axis=0)
    acc2 = jnp.dot(w2_ref[...], x2, preferred_element_type=f32)
    a2_all = jnp.tanh(acc2 + b2t_ref[...]).astype(bf16)

    pooled2 = jnp.dot(a2_all, pm_ref[...], preferred_element_type=f32)
    pooled2 = jnp.transpose(pooled2)

    p2f = jnp.concatenate(
        [pooled2[p * bb:(p + 1) * bb, :] for p in range(_FC_SP)],
        axis=1).astype(bf16)
    h = jnp.dot(p2f, wf1_ref[...], preferred_element_type=f32)
    h = jnp.tanh(h + bf1_ref[...]).astype(bf16)

    h = jnp.tanh(jnp.dot(h, wf2_ref[...], preferred_element_type=f32)
                 + bf2_ref[...]).astype(bf16)
    o = jnp.dot(h, wf3_ref[...], preferred_element_type=f32) + bf3_ref[...]
    o_ref[...] = o.reshape(bb, 1, 2)


def kernel(x, w1, b1, p1, w2, b2, p2, wf1, bf1, wf2, bf2, wf3, bf3):
    B = x.shape[0]
    bb = 1
    for cand in (16, 8, 4, 2):
        if B % cand == 0 and B // cand >= 2:
            bb = cand
            break
    bf16 = jnp.bfloat16

    xr = x.reshape(B, _CIN, _SP)

    w1r = w1.reshape(_K, _K, 8, _CONV1_N)[:, :, :_CIN, :]
    w1t = jnp.transpose(w1r, (3, 0, 1, 2)).reshape(_CONV1_N, _K, _K * _CIN)
    w1b = jnp.pad(w1t, ((0, 0), (0, 0), (0, 1))).reshape(_CONV1_N, _K * 16)
    b1s = jnp.transpose(b1)

    p1t = jnp.pad(jnp.transpose(p1[:_P1_N]), ((0, _SP - 896), (0, 0)))
    p1t = p1t.astype(bf16)

    w2t = jnp.transpose(w2)
    b2t = jnp.transpose(b2)

    eye = jnp.eye(bb, dtype=p2.dtype)
    pm = (eye[:, None, None, :] * jnp.transpose(p2)[None, :, :, None])
    pm = pm.reshape(bb * _CONV2_L, 32 * bb)

    wf1s = wf1.reshape(_FC_SP * _FC_H, _FC_H)

    body = functools.partial(_fused_body, bb=bb)
    c2 = lambda i: (0, 0)

    out = pl.pallas_call(
        body,
        grid=(B // bb,),
        out_shape=jax.ShapeDtypeStruct((B, 1, 2), jnp.float32),
        in_specs=[
            pl.BlockSpec((bb, _CIN, _SP), lambda i: (i, 0, 0)),
            pl.BlockSpec((_CONV1_N, _K * 16), c2),
            pl.BlockSpec((_CONV1_N, 1), c2),
            pl.BlockSpec((_SP, _P1_N), c2),
            pl.BlockSpec((_COUT2, _K * _K * _CONV1_N), c2),
            pl.BlockSpec((_COUT2, 1), c2),
            pl.BlockSpec((bb * _CONV2_L, 32 * bb), c2),
            pl.BlockSpec((_FC_SP * _FC_H, _FC_H), c2),
            pl.BlockSpec((1, _FC_H), c2),
            pl.BlockSpec((_FC_H, _FC_H), c2),
            pl.BlockSpec((1, _FC_H), c2),
            pl.BlockSpec((_FC_H, 2), c2),
            pl.BlockSpec((1, 2), c2),
        ],
        out_specs=pl.BlockSpec((bb, 1, 2), lambda i: (i, 0, 0)),
        compiler_params=pltpu.CompilerParams(
            dimension_semantics=("parallel",)),
    )(xr, w1b, b1s, p1t, w2t, b2t, pm,
      wf1s, bf1, wf2, bf2, wf3, bf3)
    return out.reshape(B, 2)
